# baseline (speedup 1.0000x reference)
import jax
import jax.numpy as jnp
from jax.experimental import pallas as pl

_B, _N, _C = 16, 1024, 1024
_R = 256
_G = _C * 2 // 128


def _body(x_ref, o_ref):
    j = pl.program_id(1)
    v = x_ref[0]
    lanes = jax.lax.broadcasted_iota(jnp.int32, (_R, _G, 128), 2)
    gidx = (2 * lanes + 1) & 127
    odd = jnp.take_along_axis(v, gidx, axis=2)
    rows = (
        jax.lax.broadcasted_iota(jnp.int32, (_R, _G, 128), 0) + j * _R
    ).astype(jnp.float32)
    o_ref[0] = jnp.where(odd > 0.5, rows, 0.0)[:, :, :64]


def kernel(x):
    xv = x.reshape(_B, _N, _G, 128)
    out = pl.pallas_call(
        _body,
        grid=(_B, _N // _R),
        in_specs=[pl.BlockSpec((1, _R, _G, 128), lambda b, j: (b, j, 0, 0))],
        out_specs=pl.BlockSpec((1, _R, _G, 64), lambda b, j: (b, j, 0, 0)),
        out_shape=jax.ShapeDtypeStruct((_B, _N, _G, 64), jnp.float32),
    )(xv)
    return out.reshape(_B, _N, _C)


# speedup vs baseline: 6.5535x; 6.5535x over previous
import jax
import jax.numpy as jnp
from jax.experimental import pallas as pl

_B, _N, _C = 16, 1024, 1024
_R = 256


def _body(x_ref, o_ref):
    j = pl.program_id(1)
    odd = x_ref[:, pl.Slice(1, 8, 2), :]
    v = odd.reshape(_R, _C)
    rows = (
        jax.lax.broadcasted_iota(jnp.int32, (_R, _C), 0) + j * _R
    ).astype(jnp.float32)
    o_ref[...] = jnp.where(v > 0.5, rows, 0.0)


def kernel(x):
    xt = jnp.transpose(
        x.reshape(_B, _N, _C // 128, 128, 2), (0, 1, 2, 4, 3)
    ).reshape(_B, _N, 16, 128)
    return pl.pallas_call(
        _body,
        grid=(_B, _N // _R),
        in_specs=[pl.BlockSpec((None, _R, 16, 128), lambda b, j: (b, j, 0, 0))],
        out_specs=pl.BlockSpec((None, _R, _C), lambda b, j: (b, j, 0)),
        out_shape=jax.ShapeDtypeStruct((_B, _N, _C), jnp.float32),
    )(xt)


# speedup vs baseline: 8.0053x; 1.2215x over previous
import jax
import jax.numpy as jnp
from jax.experimental import pallas as pl

_B, _N, _C = 16, 1024, 1024
_R = 512


def _body(x_ref, o_ref):
    j = pl.program_id(1)
    odd = x_ref[:, pl.Slice(1, 8, 2), :]
    v = odd.reshape(_R, _C)
    rows = (
        jax.lax.broadcasted_iota(jnp.int32, (_R, _C), 0) + j * _R
    ).astype(jnp.float32)
    o_ref[...] = jnp.where(v > 0.5, rows, 0.0)


def kernel(x):
    xt = jnp.transpose(
        x.reshape(_B, _N, _C // 128, 128, 2), (0, 1, 2, 4, 3)
    ).reshape(_B, _N, 16, 128)
    return pl.pallas_call(
        _body,
        grid=(_B, _N // _R),
        in_specs=[pl.BlockSpec((None, _R, 16, 128), lambda b, j: (b, j, 0, 0))],
        out_specs=pl.BlockSpec((None, _R, _C), lambda b, j: (b, j, 0)),
        out_shape=jax.ShapeDtypeStruct((_B, _N, _C), jnp.float32),
    )(xt)


# speedup vs baseline: 8.3487x; 1.0429x over previous
import jax
import jax.numpy as jnp
from jax.experimental import pallas as pl

_B, _N, _C = 16, 1024, 1024
_R = 1024


def _body(x_ref, o_ref):
    j = pl.program_id(1)
    odd = x_ref[:, pl.Slice(1, 8, 2), :]
    v = odd.reshape(_R, _C)
    rows = (
        jax.lax.broadcasted_iota(jnp.int32, (_R, _C), 0) + j * _R
    ).astype(jnp.float32)
    o_ref[...] = jnp.where(v > 0.5, rows, 0.0)


def kernel(x):
    xt = jnp.transpose(
        x.reshape(_B, _N, _C // 128, 128, 2), (0, 1, 2, 4, 3)
    ).reshape(_B, _N, 16, 128)
    return pl.pallas_call(
        _body,
        grid=(_B, _N // _R),
        in_specs=[pl.BlockSpec((None, _R, 16, 128), lambda b, j: (b, j, 0, 0))],
        out_specs=pl.BlockSpec((None, _R, _C), lambda b, j: (b, j, 0)),
        out_shape=jax.ShapeDtypeStruct((_B, _N, _C), jnp.float32),
    )(xt)
